# baseline (speedup 1.0000x reference)
import functools
import jax
import jax.numpy as jnp
from jax import lax
from jax.experimental import pallas as pl
from jax.experimental.pallas import tpu as pltpu
from jax.experimental.pallas import tpu_sc as plsc

_L = 16
_NW_PER_BATCH = 8
_CHUNKS = 4


def _round_bf16_rne(x):
    u = lax.bitcast_convert_type(x, jnp.uint32)
    lsb = (u >> 16) & jnp.uint32(1)
    u2 = (u + jnp.uint32(0x7FFF) + lsb) & jnp.uint32(0xFFFF0000)
    return lax.bitcast_convert_type(u2, jnp.float32)


def _make_sc_body(B, N, M):
    rows = N // _NW_PER_BATCH
    grp = _CHUNKS * _L

    def splat(v, l):
        return jnp.full((_L,), v[l], dtype=jnp.float32)

    def minpass(out_ref, ox, oy, oz, oq, sx, sy, sz, sq, n_out, n_scal,
                clamp):
        def ogroup(og, carry):
            o0 = og * grp
            oxv = [ox[pl.ds(o0 + c * _L, _L)] for c in range(_CHUNKS)]
            oyv = [oy[pl.ds(o0 + c * _L, _L)] for c in range(_CHUNKS)]
            ozv = [oz[pl.ds(o0 + c * _L, _L)] for c in range(_CHUNKS)]
            oqv = [oq[pl.ds(o0 + c * _L, _L)] for c in range(_CHUNKS)]

            def sloop(sv, accs):
                bxv = sx[pl.ds(sv * _L, _L)]
                byv = sy[pl.ds(sv * _L, _L)]
                bzv = sz[pl.ds(sv * _L, _L)]
                bqv = sq[pl.ds(sv * _L, _L)]
                accs = list(accs)
                for l in range(_L):
                    bxs = splat(bxv, l)
                    bys = splat(byv, l)
                    bzs = splat(bzv, l)
                    bqs = splat(bqv, l)
                    for c in range(_CHUNKS):
                        t = oxv[c] * bxs + oqv[c]
                        t = oyv[c] * bys + t
                        t = ozv[c] * bzs + t
                        d = t + bqs
                        accs[c] = jnp.minimum(accs[c], d)
                return tuple(accs)

            inf16 = jnp.full((_L,), jnp.inf, dtype=jnp.float32)
            accs = lax.fori_loop(0, n_scal // _L, sloop,
                                 (inf16,) * _CHUNKS)
            for c in range(_CHUNKS):
                r = accs[c]
                if clamp:
                    r = jnp.maximum(r, 0.0)
                out_ref[pl.ds(o0 + c * _L, _L)] = r
            return carry

        lax.fori_loop(0, n_out // grp, ogroup, 0)

    def body(m2a_hbm, asq_hbm, br_hbm, bsq_hbm, d1_hbm, d2_hbm,
             bx_v, by_v, bz_v, bs_v, ax_v, ay_v, az_v, aq_v,
             col_v, out_v, part_v, out2_v, shared):
        c = lax.axis_index("c")
        s = lax.axis_index("s")
        batch = c * 2 + s // _NW_PER_BATCH
        chunk = s % _NW_PER_BATCH
        base = chunk * rows

        pltpu.sync_copy(br_hbm.at[pl.ds((batch * 3 + 0) * M, M)], bx_v)
        pltpu.sync_copy(br_hbm.at[pl.ds((batch * 3 + 1) * M, M)], by_v)
        pltpu.sync_copy(br_hbm.at[pl.ds((batch * 3 + 2) * M, M)], bz_v)
        pltpu.sync_copy(bsq_hbm.at[pl.ds(batch * M, M)], bs_v)
        pltpu.sync_copy(m2a_hbm.at[pl.ds((batch * 3 + 0) * N + base, rows)], ax_v)
        pltpu.sync_copy(m2a_hbm.at[pl.ds((batch * 3 + 1) * N + base, rows)], ay_v)
        pltpu.sync_copy(m2a_hbm.at[pl.ds((batch * 3 + 2) * N + base, rows)], az_v)
        pltpu.sync_copy(asq_hbm.at[pl.ds(batch * N + base, rows)], aq_v)

        minpass(out_v, ax_v, ay_v, az_v, aq_v, bx_v, by_v, bz_v, bs_v,
                rows, M, clamp=True)
        pltpu.sync_copy(out_v, d1_hbm.at[pl.ds(batch * N + base, rows)])

        minpass(col_v, bx_v, by_v, bz_v, bs_v, ax_v, ay_v, az_v, aq_v,
                M, rows, clamp=False)

        pltpu.sync_copy(col_v, shared.at[pl.ds(s * M, M)])
        plsc.subcore_barrier()
        g0 = (s // _NW_PER_BATCH) * _NW_PER_BATCH
        for t in range(_NW_PER_BATCH):
            pltpu.sync_copy(shared.at[pl.ds((g0 + t) * M + base, rows)],
                            part_v.at[pl.ds(t * rows, rows)])

        def colred(jc, carry):
            o = jc * _L
            acc = part_v[pl.ds(o, _L)]
            for t in range(1, _NW_PER_BATCH):
                acc = jnp.minimum(acc, part_v[pl.ds(t * rows + o, _L)])
            out2_v[pl.ds(o, _L)] = jnp.maximum(acc, 0.0)
            return carry

        lax.fori_loop(0, rows // _L, colred, 0)
        pltpu.sync_copy(out2_v, d2_hbm.at[pl.ds(batch * M + base, rows)])

    return body, rows


def sc_chamfer(xyz1, xyz2):
    B, N, _ = xyz1.shape
    M = xyz2.shape[1]
    a_r = _round_bf16_rne(xyz1)
    b_r = _round_bf16_rne(xyz2)
    m2a = jnp.transpose(-2.0 * a_r, (0, 2, 1)).reshape(-1)
    br = jnp.transpose(b_r, (0, 2, 1)).reshape(-1)
    asq = jnp.sum(xyz1 * xyz1, axis=-1).reshape(-1)
    bsq = jnp.sum(xyz2 * xyz2, axis=-1).reshape(-1)

    body, rows = _make_sc_body(B, N, M)
    mesh = plsc.VectorSubcoreMesh(core_axis_name="c", subcore_axis_name="s")
    f32 = jnp.float32
    run = functools.partial(
        pl.kernel,
        out_type=[jax.ShapeDtypeStruct((B * N,), f32),
                  jax.ShapeDtypeStruct((B * M,), f32)],
        mesh=mesh,
        scratch_types=[
            pltpu.VMEM((M,), f32),
            pltpu.VMEM((M,), f32),
            pltpu.VMEM((M,), f32),
            pltpu.VMEM((M,), f32),
            pltpu.VMEM((rows,), f32),
            pltpu.VMEM((rows,), f32),
            pltpu.VMEM((rows,), f32),
            pltpu.VMEM((rows,), f32),
            pltpu.VMEM((M,), f32),
            pltpu.VMEM((rows,), f32),
            pltpu.VMEM((rows * _NW_PER_BATCH,), f32),
            pltpu.VMEM((rows,), f32),
            pltpu.VMEM_SHARED((16 * M,), f32),
        ],
    )(body)
    d1, d2 = run(m2a, asq, br, bsq)
    return d1.reshape(B, N), d2.reshape(B, M)


def kernel(xyz1, xyz2):
    return sc_chamfer(xyz1, xyz2)


# speedup vs baseline: 19.8113x; 19.8113x over previous
import jax
import jax.numpy as jnp
from jax import lax
from jax.experimental import pallas as pl
from jax.experimental.pallas import tpu as pltpu


def _split3(x):
    hi = x.astype(jnp.bfloat16)
    r = x - hi.astype(jnp.float32)
    mid = r.astype(jnp.bfloat16)
    lo = (r - mid.astype(jnp.float32)).astype(jnp.bfloat16)
    return hi, mid, lo


def _tc_chamfer_body(a_ref, b_ref, d1_ref, d2_ref, btb_ref):
    b_id = pl.program_id(0)
    i = pl.program_id(1)
    IB = a_ref.shape[1]
    M = b_ref.shape[1]

    @pl.when(i == 0)
    def _():
        b = b_ref[0]
        bt = jnp.transpose(b)
        bsq = jnp.sum(bt * bt, axis=0, keepdims=True)
        bh, bm, bl = _split3(bsq)
        one = jnp.ones((3, M), dtype=jnp.bfloat16)
        zero = jnp.zeros((7, M), dtype=jnp.bfloat16)
        btb_ref[...] = jnp.concatenate(
            [bt.astype(jnp.bfloat16), one, bh, bm, bl, zero], axis=0)

    a = -2.0 * a_ref[0]
    asq = 0.25 * jnp.sum(a * a, axis=1, keepdims=True)
    ah, am, al = _split3(asq)
    aone = jnp.ones((IB, 3), dtype=jnp.bfloat16)
    azero = jnp.zeros((IB, 7), dtype=jnp.bfloat16)
    amat = jnp.concatenate([a.astype(jnp.bfloat16), ah, am, al, aone, azero],
                           axis=1)
    d = lax.dot_general(amat, btb_ref[...], (((1,), (0,)), ((), ())),
                        preferred_element_type=jnp.float32)
    d1_ref[b_id, pl.ds(i * IB, IB)] = jnp.maximum(jnp.min(d, axis=1), 0.0)
    colpart = jnp.maximum(jnp.min(d, axis=0), 0.0)

    @pl.when(i == 0)
    def _():
        d2_ref[b_id, :] = colpart

    @pl.when(i > 0)
    def _():
        d2_ref[b_id, :] = jnp.minimum(d2_ref[b_id, :], colpart)


def kernel(xyz1, xyz2):
    B, N, _ = xyz1.shape
    M = xyz2.shape[1]
    IB = 512
    ni = N // IB
    d1, d2 = pl.pallas_call(
        _tc_chamfer_body,
        grid=(B, ni),
        in_specs=[
            pl.BlockSpec((1, IB, 3), lambda b, i: (b, i, 0)),
            pl.BlockSpec((1, M, 3), lambda b, i: (b, 0, 0)),
        ],
        out_specs=[
            pl.BlockSpec((B, N), lambda b, i: (0, 0)),
            pl.BlockSpec((B, M), lambda b, i: (0, 0)),
        ],
        out_shape=[
            jax.ShapeDtypeStruct((B, N), jnp.float32),
            jax.ShapeDtypeStruct((B, M), jnp.float32),
        ],
        scratch_shapes=[pltpu.VMEM((16, M), jnp.bfloat16)],
    )(xyz1, xyz2)
    return d1, d2
